# baseline (speedup 1.0000x reference)
import functools

import jax
import jax.numpy as jnp
from jax import lax
from jax.experimental import pallas as pl
from jax.experimental.pallas import tpu as pltpu
from jax.experimental.pallas import tpu_sc as plsc

L = 16


def kernel(pred_H_sampled, pred_batch_ix, true_index_sampled, true_H_sampled):
    B, N = pred_H_sampled.shape
    _, S = true_index_sampled.shape

    info = plsc.get_sparse_core_info()
    NC, NS = info.num_cores, info.num_subcores
    NW = NC * NS
    split = NW // B
    per_w = S // split
    R = 13
    CH = R * 128
    nch = per_w // CH
    tail = per_w - nch * CH

    mesh = plsc.VectorSubcoreMesh(core_axis_name="c", subcore_axis_name="s")

    @functools.partial(
        pl.kernel,
        mesh=mesh,
        out_type=jax.ShapeDtypeStruct((NW, L), jnp.float32),
        scratch_types=[
            pltpu.VMEM((L,), jnp.int32),
            pltpu.VMEM((CH,), jnp.int32),
            pltpu.VMEM((R, 128), jnp.int32),
            pltpu.VMEM((R, 128), jnp.float32),
            pltpu.VMEM((CH,), jnp.float32),
            pltpu.VMEM((L,), jnp.float32),
            pltpu.SemaphoreType.DMA,
        ],
    )
    def sc_body(pred_hbm, bases_hbm, idx_hbm, true_hbm, out_hbm,
                bix_v, raw_v, flat_v, vals_v, true_v, outrow_v, sem):
        c = lax.axis_index("c")
        s = lax.axis_index("s")
        w = s * NC + c
        g0 = w * per_w

        pltpu.sync_copy(bases_hbm.at[w], bix_v)
        base = bix_v[...]

        def chunk(ci, acc):
            off = g0 + ci * CH
            pltpu.sync_copy(idx_hbm.at[pl.ds(off, CH)], raw_v)
            pltpu.sync_copy(true_hbm.at[pl.ds(off, CH)], true_v)
            for r in range(R):
                for j in range(8):
                    flat_v[r, pl.ds(j * L, L)] = (
                        raw_v[pl.ds(r * 128 + j * L, L)] + base
                    )
            copies = [
                pltpu.async_copy(pred_hbm.at[flat_v.at[r]], vals_v.at[r], sem)
                for r in range(R)
            ]
            for cp in copies:
                cp.wait()
            for r in range(R):
                for j in range(8):
                    g = vals_v[r, pl.ds(j * L, L)]
                    t = true_v[pl.ds(r * 128 + j * L, L)]
                    d = g - t
                    acc = acc + d * d
            return acc

        acc = lax.fori_loop(0, nch, chunk, jnp.zeros((L,), jnp.float32))

        if tail:
            off = g0 + nch * CH
            for j in range(8):
                flat_v[0, pl.ds(j * L, L)] = base
            pltpu.sync_copy(idx_hbm.at[pl.ds(off, tail)],
                            raw_v.at[pl.ds(0, tail)])
            pltpu.sync_copy(true_hbm.at[pl.ds(off, tail)],
                            true_v.at[pl.ds(0, tail)])
            lanes = lax.iota(jnp.int32, L)
            nj = (tail + L - 1) // L
            for j in range(nj):
                nvalid = min(L, tail - j * L)
                m = lanes < nvalid
                raw = raw_v[pl.ds(j * L, L)]
                flat_v[0, pl.ds(j * L, L)] = jnp.where(m, raw + base, base)
            pltpu.async_copy(pred_hbm.at[flat_v.at[0]], vals_v.at[0], sem).wait()
            for j in range(nj):
                nvalid = min(L, tail - j * L)
                m = lanes < nvalid
                g = vals_v[0, pl.ds(j * L, L)]
                t = true_v[pl.ds(j * L, L)]
                d = jnp.where(m, g - t, jnp.float32(0.0))
                acc = acc + d * d

        outrow_v[...] = acc
        pltpu.sync_copy(outrow_v, out_hbm.at[w])

    pred_flat = pred_H_sampled.reshape(-1)
    row_base = pred_batch_ix.astype(jnp.int32) * jnp.int32(N)
    bases = jnp.broadcast_to(
        jnp.repeat(row_base, split)[:, None], (NW, L)
    )
    idx_flat = true_index_sampled.astype(jnp.int32).reshape(-1)
    true_flat = true_H_sampled.reshape(-1)
    partials = sc_body(pred_flat, bases, idx_flat, true_flat)
    return partials.sum() / (B * S)


# speedup vs baseline: 1.1538x; 1.1538x over previous
import functools

import jax
import jax.numpy as jnp
from jax import lax
from jax.experimental import pallas as pl
from jax.experimental.pallas import tpu as pltpu
from jax.experimental.pallas import tpu_sc as plsc

L = 16


def kernel(pred_H_sampled, pred_batch_ix, true_index_sampled, true_H_sampled):
    B, N = pred_H_sampled.shape
    _, S = true_index_sampled.shape

    info = plsc.get_sparse_core_info()
    NC, NS = info.num_cores, info.num_subcores
    NW = NC * NS
    split = NW // B
    per_w = S // split
    R = 13
    CH = R * 128
    nch = per_w // CH
    tail = per_w - nch * CH

    mesh = plsc.VectorSubcoreMesh(core_axis_name="c", subcore_axis_name="s")

    @functools.partial(
        pl.kernel,
        mesh=mesh,
        out_type=jax.ShapeDtypeStruct((NW, L), jnp.float32),
        scratch_types=[
            pltpu.VMEM((L,), jnp.int32),
            pltpu.VMEM((CH,), jnp.int32),
            pltpu.VMEM((CH,), jnp.int32),
            pltpu.VMEM((R, 128), jnp.int32),
            pltpu.VMEM((R, 128), jnp.int32),
            pltpu.VMEM((R, 128), jnp.float32),
            pltpu.VMEM((R, 128), jnp.float32),
            pltpu.VMEM((CH,), jnp.float32),
            pltpu.VMEM((CH,), jnp.float32),
            pltpu.VMEM((L,), jnp.float32),
            pltpu.SemaphoreType.DMA,
            pltpu.SemaphoreType.DMA,
            pltpu.SemaphoreType.DMA,
        ],
    )
    def sc_body(pred_hbm, bases_hbm, idx_hbm, true_hbm, out_hbm,
                bix_v, raw0_v, raw1_v, flat0_v, flat1_v, vals0_v, vals1_v,
                true0_v, true1_v, outrow_v, sem_i, sem_t, sem_g):
        raws = [raw0_v, raw1_v]
        flats = [flat0_v, flat1_v]
        valss = [vals0_v, vals1_v]
        trues = [true0_v, true1_v]
        c = lax.axis_index("c")
        s = lax.axis_index("s")
        w = s * NC + c
        g0 = w * per_w

        pltpu.sync_copy(bases_hbm.at[w], bix_v)
        base = bix_v[...]

        def fire(cc, buf):
            off = g0 + cc * CH
            pltpu.async_copy(idx_hbm.at[pl.ds(off, CH)], raws[buf], sem_i)
            pltpu.async_copy(true_hbm.at[pl.ds(off, CH)], trues[buf], sem_t)

        def process(cc, buf, acc, fire_c=None, fire_pred=False):
            raw_v, flat_v, vals_v, true_v = (
                raws[buf], flats[buf], valss[buf], trues[buf])
            off = g0 + cc * CH
            pltpu.make_async_copy(idx_hbm.at[pl.ds(off, CH)], raw_v,
                                  sem_i).wait()
            for r in range(R):
                for j in range(8):
                    flat_v[r, pl.ds(j * L, L)] = (
                        raw_v[pl.ds(r * 128 + j * L, L)] + base
                    )
            gs = [
                pltpu.async_copy(pred_hbm.at[flat_v.at[r]], vals_v.at[r],
                                 sem_g)
                for r in range(R)
            ]
            if fire_c is not None:
                foff = g0 + fire_c * CH
                if fire_pred:
                    @pl.when(fire_c < nch)
                    def _():
                        pltpu.async_copy(idx_hbm.at[pl.ds(foff, CH)],
                                         raw_v, sem_i)
                else:
                    pltpu.async_copy(idx_hbm.at[pl.ds(foff, CH)], raw_v,
                                     sem_i)
            pltpu.make_async_copy(true_hbm.at[pl.ds(off, CH)], true_v,
                                  sem_t).wait()
            for r in range(R):
                gs[r].wait()
                for j in range(8):
                    g = vals_v[r, pl.ds(j * L, L)]
                    t = true_v[pl.ds(r * 128 + j * L, L)]
                    d = g - t
                    acc = acc + d * d
            if fire_c is not None:
                foff = g0 + fire_c * CH
                if fire_pred:
                    @pl.when(fire_c < nch)
                    def _():
                        pltpu.async_copy(true_hbm.at[pl.ds(foff, CH)],
                                         true_v, sem_t)
                else:
                    pltpu.async_copy(true_hbm.at[pl.ds(foff, CH)], true_v,
                                     sem_t)
            return acc

        fire(0, 0)
        fire(1, 1)

        def loop_body(i, acc):
            c0 = 2 * i
            acc = process(c0, 0, acc, fire_c=c0 + 2)
            acc = process(c0 + 1, 1, acc, fire_c=c0 + 3, fire_pred=True)
            return acc

        acc = lax.fori_loop(0, (nch - 1) // 2, loop_body,
                            jnp.zeros((L,), jnp.float32))
        acc = process(nch - 1, 0, acc)

        if tail:
            off = g0 + nch * CH
            for j in range(8):
                flat0_v[0, pl.ds(j * L, L)] = base
            pltpu.sync_copy(idx_hbm.at[pl.ds(off, tail)],
                            raw0_v.at[pl.ds(0, tail)])
            pltpu.sync_copy(true_hbm.at[pl.ds(off, tail)],
                            true0_v.at[pl.ds(0, tail)])
            lanes = lax.iota(jnp.int32, L)
            nj = (tail + L - 1) // L
            for j in range(nj):
                nvalid = min(L, tail - j * L)
                m = lanes < nvalid
                raw = raw0_v[pl.ds(j * L, L)]
                flat0_v[0, pl.ds(j * L, L)] = jnp.where(m, raw + base, base)
            pltpu.async_copy(pred_hbm.at[flat0_v.at[0]], vals0_v.at[0],
                             sem_g).wait()
            for j in range(nj):
                nvalid = min(L, tail - j * L)
                m = lanes < nvalid
                g = vals0_v[0, pl.ds(j * L, L)]
                t = true0_v[pl.ds(j * L, L)]
                d = jnp.where(m, g - t, jnp.float32(0.0))
                acc = acc + d * d

        outrow_v[...] = acc
        pltpu.sync_copy(outrow_v, out_hbm.at[w])

    pred_flat = pred_H_sampled.reshape(-1)
    row_base = pred_batch_ix.astype(jnp.int32) * jnp.int32(N)
    bases = jnp.broadcast_to(
        jnp.repeat(row_base, split)[:, None], (NW, L)
    )
    idx_flat = true_index_sampled.astype(jnp.int32).reshape(-1)
    true_flat = true_H_sampled.reshape(-1)
    partials = sc_body(pred_flat, bases, idx_flat, true_flat)
    return partials.sum() / (B * S)


# speedup vs baseline: 1.2502x; 1.0835x over previous
import functools

import jax
import jax.numpy as jnp
from jax import lax
from jax.experimental import pallas as pl
from jax.experimental.pallas import tpu as pltpu
from jax.experimental.pallas import tpu_sc as plsc

L = 16


def kernel(pred_H_sampled, pred_batch_ix, true_index_sampled, true_H_sampled):
    B, N = pred_H_sampled.shape
    _, S = true_index_sampled.shape

    info = plsc.get_sparse_core_info()
    NC, NS = info.num_cores, info.num_subcores
    NW = NC * NS
    split = NW // B
    per_w = S // split
    R = 13
    CH = R * 128
    nch = per_w // CH
    tail = per_w - nch * CH

    mesh = plsc.VectorSubcoreMesh(core_axis_name="c", subcore_axis_name="s")

    @functools.partial(
        pl.kernel,
        mesh=mesh,
        out_type=jax.ShapeDtypeStruct((NW, L), jnp.float32),
        scratch_types=[
            pltpu.VMEM((L,), jnp.int32),
            pltpu.VMEM((CH,), jnp.int32),
            pltpu.VMEM((CH,), jnp.int32),
            pltpu.VMEM((R, 128), jnp.int32),
            pltpu.VMEM((R, 128), jnp.int32),
            pltpu.VMEM((R, 128), jnp.float32),
            pltpu.VMEM((R, 128), jnp.float32),
            pltpu.VMEM((CH,), jnp.float32),
            pltpu.VMEM((CH,), jnp.float32),
            pltpu.VMEM((L,), jnp.float32),
            pltpu.SemaphoreType.DMA,
            pltpu.SemaphoreType.DMA,
            pltpu.SemaphoreType.DMA,
        ],
    )
    def sc_body(pred_hbm, bases_hbm, idx_hbm, true_hbm, out_hbm,
                bix_v, raw0_v, raw1_v, flat0_v, flat1_v, vals0_v, vals1_v,
                true0_v, true1_v, outrow_v, sem_i, sem_t, sem_g):
        raws = [raw0_v, raw1_v]
        flats = [flat0_v, flat1_v]
        valss = [vals0_v, vals1_v]
        trues = [true0_v, true1_v]
        c = lax.axis_index("c")
        s = lax.axis_index("s")
        w = s * NC + c
        g0 = w * per_w

        pltpu.sync_copy(bases_hbm.at[w], bix_v)
        base = bix_v[...]

        def stage_a(cc, buf, fire_idx_c=None, fire_pred=False):
            raw_v, flat_v, vals_v = raws[buf], flats[buf], valss[buf]
            off = g0 + cc * CH
            pltpu.make_async_copy(idx_hbm.at[pl.ds(off, CH)], raw_v,
                                  sem_i).wait()
            for r in range(R):
                for j in range(8):
                    flat_v[r, pl.ds(j * L, L)] = (
                        raw_v[pl.ds(r * 128 + j * L, L)] + base
                    )
            for r in range(R):
                pltpu.async_copy(pred_hbm.at[flat_v.at[r]], vals_v.at[r],
                                 sem_g)
            if fire_idx_c is not None:
                foff = g0 + fire_idx_c * CH
                if fire_pred:
                    @pl.when(fire_idx_c < nch)
                    def _():
                        pltpu.async_copy(idx_hbm.at[pl.ds(foff, CH)],
                                         raw_v, sem_i)
                else:
                    pltpu.async_copy(idx_hbm.at[pl.ds(foff, CH)], raw_v,
                                     sem_i)

        def stage_b(cc, buf, acc, fire_true_c=None, fire_pred=False):
            flat_v, vals_v, true_v = flats[buf], valss[buf], trues[buf]
            off = g0 + cc * CH
            pltpu.make_async_copy(true_hbm.at[pl.ds(off, CH)], true_v,
                                  sem_t).wait()
            for r in range(R):
                pltpu.make_async_copy(pred_hbm.at[flat_v.at[r]],
                                      vals_v.at[r], sem_g).wait()
                for j in range(8):
                    g = vals_v[r, pl.ds(j * L, L)]
                    t = true_v[pl.ds(r * 128 + j * L, L)]
                    d = g - t
                    acc = acc + d * d
            if fire_true_c is not None:
                foff = g0 + fire_true_c * CH
                if fire_pred:
                    @pl.when(fire_true_c < nch)
                    def _():
                        pltpu.async_copy(true_hbm.at[pl.ds(foff, CH)],
                                         true_v, sem_t)
                else:
                    pltpu.async_copy(true_hbm.at[pl.ds(foff, CH)], true_v,
                                     sem_t)
            return acc

        pltpu.async_copy(idx_hbm.at[pl.ds(g0, CH)], raw0_v, sem_i)
        pltpu.async_copy(idx_hbm.at[pl.ds(g0 + CH, CH)], raw1_v, sem_i)
        pltpu.async_copy(true_hbm.at[pl.ds(g0, CH)], true0_v, sem_t)
        pltpu.async_copy(true_hbm.at[pl.ds(g0 + CH, CH)], true1_v, sem_t)
        stage_a(0, 0, fire_idx_c=2 if nch > 2 else None)

        def loop_body(i, acc):
            c0 = 2 * i
            stage_a(c0 + 1, 1, fire_idx_c=c0 + 3, fire_pred=True)
            acc = stage_b(c0, 0, acc, fire_true_c=c0 + 2, fire_pred=True)
            stage_a(c0 + 2, 0, fire_idx_c=c0 + 4, fire_pred=True)
            acc = stage_b(c0 + 1, 1, acc, fire_true_c=c0 + 3, fire_pred=True)
            return acc

        acc = lax.fori_loop(0, (nch - 1) // 2, loop_body,
                            jnp.zeros((L,), jnp.float32))
        acc = stage_b(nch - 1, 0, acc)

        if tail:
            off = g0 + nch * CH
            for j in range(8):
                flat0_v[0, pl.ds(j * L, L)] = base
            pltpu.sync_copy(idx_hbm.at[pl.ds(off, tail)],
                            raw0_v.at[pl.ds(0, tail)])
            pltpu.sync_copy(true_hbm.at[pl.ds(off, tail)],
                            true0_v.at[pl.ds(0, tail)])
            lanes = lax.iota(jnp.int32, L)
            nj = (tail + L - 1) // L
            for j in range(nj):
                nvalid = min(L, tail - j * L)
                m = lanes < nvalid
                raw = raw0_v[pl.ds(j * L, L)]
                flat0_v[0, pl.ds(j * L, L)] = jnp.where(m, raw + base, base)
            pltpu.async_copy(pred_hbm.at[flat0_v.at[0]], vals0_v.at[0],
                             sem_g).wait()
            for j in range(nj):
                nvalid = min(L, tail - j * L)
                m = lanes < nvalid
                g = vals0_v[0, pl.ds(j * L, L)]
                t = true0_v[pl.ds(j * L, L)]
                d = jnp.where(m, g - t, jnp.float32(0.0))
                acc = acc + d * d

        outrow_v[...] = acc
        pltpu.sync_copy(outrow_v, out_hbm.at[w])

    pred_flat = pred_H_sampled.reshape(-1)
    row_base = pred_batch_ix.astype(jnp.int32) * jnp.int32(N)
    bases = jnp.broadcast_to(
        jnp.repeat(row_base, split)[:, None], (NW, L)
    )
    idx_flat = true_index_sampled.astype(jnp.int32).reshape(-1)
    true_flat = true_H_sampled.reshape(-1)
    partials = sc_body(pred_flat, bases, idx_flat, true_flat)
    return partials.sum() / (B * S)


# speedup vs baseline: 1.3127x; 1.0500x over previous
import functools

import jax
import jax.numpy as jnp
from jax import lax
from jax.experimental import pallas as pl
from jax.experimental.pallas import tpu as pltpu
from jax.experimental.pallas import tpu_sc as plsc

L = 16


def kernel(pred_H_sampled, pred_batch_ix, true_index_sampled, true_H_sampled):
    B, N = pred_H_sampled.shape
    _, S = true_index_sampled.shape

    info = plsc.get_sparse_core_info()
    NC, NS = info.num_cores, info.num_subcores
    NW = NC * NS
    split = NW // B
    per_w = S // split
    CH = 2272
    nch = per_w // CH
    tail = per_w - nch * CH
    assert tail <= L and nch % 2 == 1 and CH % L == 0

    mesh = plsc.VectorSubcoreMesh(core_axis_name="c", subcore_axis_name="s")

    @functools.partial(
        pl.kernel,
        mesh=mesh,
        out_type=jax.ShapeDtypeStruct((NW, L), jnp.float32),
        scratch_types=[
            pltpu.VMEM((L,), jnp.int32),
            pltpu.VMEM((CH,), jnp.int32),
            pltpu.VMEM((CH,), jnp.int32),
            pltpu.VMEM((CH,), jnp.int32),
            pltpu.VMEM((CH,), jnp.int32),
            pltpu.VMEM((CH,), jnp.float32),
            pltpu.VMEM((CH,), jnp.float32),
            pltpu.VMEM((CH,), jnp.float32),
            pltpu.VMEM((CH,), jnp.float32),
            pltpu.VMEM((L,), jnp.int32),
            pltpu.VMEM((L,), jnp.float32),
            pltpu.VMEM((L,), jnp.float32),
            pltpu.SemaphoreType.DMA,
            pltpu.SemaphoreType.DMA,
            pltpu.SemaphoreType.DMA,
        ],
    )
    def sc_body(pred_hbm, bases_hbm, idx_hbm, true_hbm, out_hbm,
                bix_v, raw0_v, raw1_v, flat0_v, flat1_v, vals0_v, vals1_v,
                true0_v, true1_v, tailidx_v, tailvals_v, outrow_v,
                sem_i, sem_t, sem_g):
        raws = [raw0_v, raw1_v]
        flats = [flat0_v, flat1_v]
        valss = [vals0_v, vals1_v]
        trues = [true0_v, true1_v]
        c = lax.axis_index("c")
        s = lax.axis_index("s")
        w = s * NC + c
        g0 = w * per_w

        pltpu.sync_copy(bases_hbm.at[w], bix_v)
        base = bix_v[...]

        def stage_a(cc, buf, fire_idx_c=None, fire_pred=False):
            raw_v, flat_v, vals_v = raws[buf], flats[buf], valss[buf]
            off = g0 + cc * CH
            pltpu.make_async_copy(idx_hbm.at[pl.ds(off, CH)], raw_v,
                                  sem_i).wait()
            for k in range(CH // L):
                flat_v[pl.ds(k * L, L)] = raw_v[pl.ds(k * L, L)] + base
            pltpu.async_copy(pred_hbm.at[flat_v], vals_v, sem_g)
            if fire_idx_c is not None:
                foff = g0 + fire_idx_c * CH
                if fire_pred:
                    @pl.when(fire_idx_c < nch)
                    def _():
                        pltpu.async_copy(idx_hbm.at[pl.ds(foff, CH)],
                                         raw_v, sem_i)
                else:
                    pltpu.async_copy(idx_hbm.at[pl.ds(foff, CH)], raw_v,
                                     sem_i)

        def stage_b(cc, buf, acc, fire_true_c=None, fire_pred=False):
            flat_v, vals_v, true_v = flats[buf], valss[buf], trues[buf]
            off = g0 + cc * CH
            pltpu.make_async_copy(true_hbm.at[pl.ds(off, CH)], true_v,
                                  sem_t).wait()
            pltpu.make_async_copy(pred_hbm.at[flat_v], vals_v, sem_g).wait()
            for k in range(CH // L):
                g = vals_v[pl.ds(k * L, L)]
                t = true_v[pl.ds(k * L, L)]
                d = g - t
                acc = acc + d * d
            if fire_true_c is not None:
                foff = g0 + fire_true_c * CH
                if fire_pred:
                    @pl.when(fire_true_c < nch)
                    def _():
                        pltpu.async_copy(true_hbm.at[pl.ds(foff, CH)],
                                         true_v, sem_t)
                else:
                    pltpu.async_copy(true_hbm.at[pl.ds(foff, CH)], true_v,
                                     sem_t)
            return acc

        pltpu.async_copy(idx_hbm.at[pl.ds(g0, CH)], raw0_v, sem_i)
        pltpu.async_copy(idx_hbm.at[pl.ds(g0 + CH, CH)], raw1_v, sem_i)
        pltpu.async_copy(true_hbm.at[pl.ds(g0, CH)], true0_v, sem_t)
        pltpu.async_copy(true_hbm.at[pl.ds(g0 + CH, CH)], true1_v, sem_t)
        stage_a(0, 0, fire_idx_c=2 if nch > 2 else None)

        def loop_body(i, acc):
            c0 = 2 * i
            stage_a(c0 + 1, 1, fire_idx_c=c0 + 3, fire_pred=True)
            acc = stage_b(c0, 0, acc, fire_true_c=c0 + 2, fire_pred=True)
            stage_a(c0 + 2, 0, fire_idx_c=c0 + 4, fire_pred=True)
            acc = stage_b(c0 + 1, 1, acc, fire_true_c=c0 + 3, fire_pred=True)
            return acc

        acc = lax.fori_loop(0, (nch - 1) // 2, loop_body,
                            jnp.zeros((L,), jnp.float32))
        acc = stage_b(nch - 1, 0, acc)

        if tail:
            off = g0 + nch * CH
            pltpu.sync_copy(idx_hbm.at[pl.ds(off, tail)],
                            raw0_v.at[pl.ds(0, tail)])
            pltpu.sync_copy(true_hbm.at[pl.ds(off, tail)],
                            true0_v.at[pl.ds(0, tail)])
            lanes = lax.iota(jnp.int32, L)
            m = lanes < tail
            raw = raw0_v[pl.ds(0, L)]
            tailidx_v[...] = jnp.where(m, raw + base, base)
            pltpu.async_copy(pred_hbm.at[tailidx_v], tailvals_v,
                             sem_g).wait()
            g = tailvals_v[...]
            t = true0_v[pl.ds(0, L)]
            d = jnp.where(m, g - t, jnp.float32(0.0))
            acc = acc + d * d

        outrow_v[...] = acc
        pltpu.sync_copy(outrow_v, out_hbm.at[w])

    pred_flat = pred_H_sampled.reshape(-1)
    row_base = pred_batch_ix.astype(jnp.int32) * jnp.int32(N)
    bases = jnp.broadcast_to(
        jnp.repeat(row_base, split)[:, None], (NW, L)
    )
    idx_flat = true_index_sampled.astype(jnp.int32).reshape(-1)
    true_flat = true_H_sampled.reshape(-1)
    partials = sc_body(pred_flat, bases, idx_flat, true_flat)
    return partials.sum() / (B * S)
